# baseline (speedup 1.0000x reference)
import functools

import jax
import jax.numpy as jnp
from jax import lax
from jax.experimental import pallas as pl
from jax.experimental.pallas import tpu as pltpu
from jax.experimental.pallas import tpu_sc as plsc

NC = 2
NS = 16
NW = NC * NS
L = 16
C = 128
NB = 4
NT = 2
NK2 = 4

BATCH = 16384
HIST = 50
D = 32

NCHUNK = BATCH * HIST // (NW * C)
NTILES = BATCH * HIST * D // (8 * 128)
NTPW = NTILES // NW


@functools.lru_cache(maxsize=None)
def _build_gather():
    mesh = plsc.VectorSubcoreMesh(core_axis_name="c", subcore_axis_name="s")

    @functools.partial(
        pl.kernel,
        mesh=mesh,
        out_type=jax.ShapeDtypeStruct((HIST * D // 8, BATCH // C, 8, C),
                                      jnp.float32),
        scratch_types=[
            pltpu.VMEM((NCHUNK, C), jnp.int32),
            pltpu.VMEM((NB, C, D), jnp.float32),
            pltpu.VMEM((NT, 4, 1, 8, C), jnp.float32),
        ] + [pltpu.SemaphoreType.DMA] * (NB + NT),
        compiler_params=pltpu.CompilerParams(
            use_tc_tiling_on_sc=False, needs_layout_passes=False),
    )
    def gather_kernel(tab_hbm, idx_hbm, out_hbm, idx_v, rows_v, t_v, *sems):
        gsems, wsems = sems[:NB], sems[NB:]
        w = lax.axis_index("s") * NC + lax.axis_index("c")
        pltpu.sync_copy(idx_hbm.at[pl.ds(w * NCHUNK, NCHUNK)], idx_v)

        def start_gather(c, b):
            pltpu.async_copy(tab_hbm.at[idx_v.at[c]], rows_v.at[b], gsems[b])

        def wait_gather(c, b):
            pltpu.make_async_copy(
                tab_hbm.at[idx_v.at[c]], rows_v.at[b], gsems[b]).wait()

        def out_slice(c):
            u = w * NCHUNK + c
            h, tb = u // (BATCH // C), u % (BATCH // C)
            return out_hbm.at[pl.ds(h * 4, 4), pl.ds(tb, 1)]

        def wait_write(c, st):
            pltpu.make_async_copy(t_v.at[st], out_slice(c), wsems[st]).wait()

        for b in range(NB):
            start_gather(b, b)

        def step(c, b, st, first, last):
            wait_gather(c, b)
            if not first:
                wait_write(c - NT, st)
            for g in range(C // L):
                lvec = lax.iota(jnp.int32, L) + g * L
                for tr in range(4):
                    for s in range(8):
                        dvec = jnp.full((L,), 8 * tr + s, jnp.int32)
                        vals = plsc.load_gather(rows_v.at[b], [lvec, dvec])
                        t_v[st, tr, 0, s, pl.ds(g * L, L)] = vals
            pltpu.async_copy(t_v.at[st], out_slice(c), wsems[st])
            if not last:
                start_gather(c + NB, b)

        def body(m, carry):
            c0 = m * NB
            for b in range(NB):
                step(c0 + b, b, b % NT, False, False)
            return carry

        for b in range(NB):
            step(b, b, b % NT, b < NT, False)
        lax.fori_loop(1, NCHUNK // NB - 1, body, 0)
        c0 = NCHUNK - NB
        for b in range(NB):
            step(c0 + b, b, b % NT, False, True)
        for st in range(NT):
            wait_write(NCHUNK - NT + st, (NCHUNK - NT + st) % NT)

    return gather_kernel


@functools.lru_cache(maxsize=None)
def _build_detile():
    mesh = plsc.VectorSubcoreMesh(core_axis_name="c", subcore_axis_name="s")

    @functools.partial(
        pl.kernel,
        mesh=mesh,
        out_type=jax.ShapeDtypeStruct((HIST, D, BATCH), jnp.float32),
        scratch_types=[pltpu.SemaphoreType.DMA] * NK2,
        compiler_params=pltpu.CompilerParams(use_tc_tiling_on_sc=True),
    )
    def detile_kernel(in_hbm, out_hbm, *sems):
        w = lax.axis_index("s") * NC + lax.axis_index("c")

        def refs(j):
            g = w * NTPW + j
            h = g // (4 * (BATCH // C))
            tr = (g // (BATCH // C)) % 4
            tb = g % (BATCH // C)
            src = in_hbm.at[pl.ds(g, 1)]
            dst = out_hbm.at[pl.ds(h, 1), pl.ds(tr * 8, 8), pl.ds(tb * C, C)]
            return src, dst

        def start(j, b):
            src, dst = refs(j)
            pltpu.async_copy(src, dst, sems[b])

        def wait(j, b):
            src, dst = refs(j)
            pltpu.make_async_copy(src, dst, sems[b]).wait()

        for b in range(NK2):
            start(b, b)

        def body(m, carry):
            j0 = m * NK2
            for b in range(NK2):
                wait(j0 + b, b)
                start(j0 + b + NK2, b)
            return carry

        lax.fori_loop(0, NTPW // NK2 - 1, body, 0)
        j0 = NTPW - NK2
        for b in range(NK2):
            wait(j0 + b, b)

    return detile_kernel


def kernel(inputs, embeddings):
    idxF = inputs.astype(jnp.int32).T.reshape(HIST * BATCH // C, C)
    mid = _build_gather()(embeddings, idxF)
    mid3 = mid.reshape(NTILES, 8, C)
    out = _build_detile()(mid3)
    return jnp.transpose(out, (2, 0, 1))
